# baseline (speedup 1.0000x reference)
import functools
import jax
import jax.numpy as jnp
from jax import lax
from jax.experimental import pallas as pl
from jax.experimental.pallas import tpu as pltpu
from jax.experimental.pallas import tpu_sc as plsc

NC = 2
NS = 16
NW = NC * NS
L = 16
G = 128
BLK = 1024


def _lane_bcast(v, jj):
  dnums = lax.GatherDimensionNumbers(
      offset_dims=(), collapsed_slice_dims=(0,), start_index_map=(0,))
  idx = jnp.full((L, 1), jj, jnp.int32)
  return lax.gather(v, idx, dnums, slice_sizes=(1,),
                    mode=lax.GatherScatterMode.PROMISE_IN_BOUNDS)


def _sc_mesh():
  return plsc.VectorSubcoreMesh(
      core_axis_name="c", subcore_axis_name="s", num_cores=NC,
      num_subcores=NS)


def _make_deg_kernel(Np, NGT):
  ng = NGT // 2

  @functools.partial(
      pl.kernel,
      out_type=jax.ShapeDtypeStruct((NC, Np), jnp.float32),
      mesh=_sc_mesh(),
      scratch_types=[
          pltpu.VMEM((ng, G), jnp.int32),
          pltpu.VMEM((ng, G), jnp.float32),
          pltpu.VMEM((Np // NS,), jnp.float32),
          pltpu.VMEM_SHARED((Np,), jnp.float32),
      ],
  )
  def deg_kernel(col_hbm, w_hbm, out_hbm, colv, wv, tmpv, deg_sh):
    cid = lax.axis_index("c")
    sid = lax.axis_index("s")
    rows = Np // NS
    base = sid * rows
    goff = cid * ng

    @pl.loop(0, rows // L)
    def _zero(i):
      tmpv[pl.ds(i * L, L)] = jnp.zeros((L,), jnp.float32)

    pltpu.sync_copy(tmpv, deg_sh.at[pl.ds(base, rows)])
    plsc.subcore_barrier()

    pltpu.sync_copy(col_hbm.at[sid, pl.ds(goff, ng)], colv)
    pltpu.sync_copy(w_hbm.at[sid, pl.ds(goff, ng)], wv)

    @pl.loop(0, ng)
    def _acc(g):
      pltpu.sync_copy(wv.at[g], deg_sh.at[colv.at[g]], add=True)

    plsc.subcore_barrier()
    pltpu.sync_copy(deg_sh.at[pl.ds(base, rows)], tmpv)
    pltpu.sync_copy(tmpv, out_hbm.at[cid, pl.ds(base, rows)])

  return deg_kernel


def _make_agg_kernel(Np, NGT, D, col_split, cache_src=True):
  HD = D // 2 if col_split else D
  ZR = 64
  ng = NGT if col_split else NGT // 2
  assert ng % 2 == 0 and ng >= 4

  scratch = [
      pltpu.VMEM((2, 2, G), jnp.int32),
      pltpu.VMEM((2, G), jnp.float32),
      pltpu.VMEM((2, G, HD), jnp.float32),
      pltpu.VMEM((ZR, HD), jnp.float32),
      pltpu.VMEM_SHARED((Np, HD), jnp.float32),
  ]
  if cache_src:
    scratch.append(pltpu.VMEM_SHARED((Np, HD), jnp.float32))
  scratch += [pltpu.SemaphoreType.DMA] * 4

  @functools.partial(
      pl.kernel,
      out_type=jax.ShapeDtypeStruct((NC, Np, HD), jnp.float32),
      mesh=_sc_mesh(),
      scratch_types=scratch,
      compiler_params=pltpu.CompilerParams(use_tc_tiling_on_sc=False),
  )
  def agg_kernel(h_hbm, idx_hbm, w_hbm, out_hbm, cbuf, wbuf, rbuf, zv,
                 acc_sh, *rest):
    if cache_src:
      src_sh = rest[0]
      semi0, semi1, semg0, semg1 = rest[1:]
    else:
      src_sh = h_hbm
      semi0, semi1, semg0, semg1 = rest
    cid = lax.axis_index("c")
    sid = lax.axis_index("s")
    rows = Np // NS
    base = sid * rows
    goff = 0 if col_split else cid * ng
    semi = (semi0, semi1)
    semg = (semg0, semg1)

    if cache_src:
      for j in range(rows // ZR):
        sl = pl.ds(base + j * ZR, ZR)
        if col_split:
          pltpu.sync_copy(h_hbm.at[cid, sl], zv)
        else:
          pltpu.sync_copy(h_hbm.at[sl], zv)
        pltpu.sync_copy(zv, src_sh.at[sl])

    @pl.loop(0, ZR)
    def _zrow(r):
      for c in range(HD // L):
        zv[r, pl.ds(c * L, L)] = jnp.zeros((L,), jnp.float32)

    for j in range(rows // ZR):
      pltpu.sync_copy(zv, acc_sh.at[pl.ds(base + j * ZR, ZR)])
    plsc.subcore_barrier()

    def idx_start(g, b):
      pltpu.async_copy(idx_hbm.at[sid, g], cbuf.at[b], semi[b])
      pltpu.async_copy(w_hbm.at[sid, g], wbuf.at[b], semi[b])

    def idx_wait(g, b):
      pltpu.make_async_copy(idx_hbm.at[sid, g], cbuf.at[b], semi[b]).wait()
      pltpu.make_async_copy(w_hbm.at[sid, g], wbuf.at[b], semi[b]).wait()

    def gather_start(b):
      pltpu.async_copy(src_sh.at[cbuf.at[b, 0]], rbuf.at[b], semg[b])

    def gather_wait(b):
      pltpu.make_async_copy(src_sh.at[cbuf.at[b, 0]], rbuf.at[b],
                            semg[b]).wait()

    def process(b):
      gather_wait(b)

      @pl.loop(0, G // L)
      def _e16(t):
        vw = wbuf[b, pl.ds(t * L, L)]
        for jj in range(L):
          sv = _lane_bcast(vw, jj)
          r = t * L + jj
          for c in range(HD // L):
            rbuf[b, r, pl.ds(c * L, L)] = rbuf[b, r, pl.ds(c * L, L)] * sv

      pltpu.sync_copy(rbuf.at[b], acc_sh.at[cbuf.at[b, 1]], add=True)

    idx_start(goff + 0, 0)
    idx_wait(goff + 0, 0)
    gather_start(0)
    idx_start(goff + 1, 1)
    idx_wait(goff + 1, 1)

    @pl.loop(0, ng - 2, step=2)
    def _pipe(g):
      gather_start(1)
      process(0)
      idx_start(goff + g + 2, 0)
      idx_wait(goff + g + 2, 0)
      gather_start(0)
      process(1)
      idx_start(goff + g + 3, 1)
      idx_wait(goff + g + 3, 1)

    gather_start(1)
    process(0)
    process(1)

    plsc.subcore_barrier()
    for j in range(rows // ZR):
      sl = pl.ds(base + j * ZR, ZR)
      pltpu.sync_copy(acc_sh.at[sl], zv)
      pltpu.sync_copy(zv, out_hbm.at[cid, sl])

  return agg_kernel


def _tc1_body(x_ref, w1_ref, degt_ref, h_ref, dinv_ref):
  deg = jnp.sum(degt_ref[...], axis=1, keepdims=True)
  dinv = jnp.where(deg > 0, lax.rsqrt(deg), 0.0)
  h = jnp.dot(x_ref[...], w1_ref[...], preferred_element_type=jnp.float32)
  h_ref[...] = h * dinv
  dinv_ref[...] = dinv


def _tc2_body(p0_ref, p1_ref, dinv_ref, b1_ref, w2_ref, h2_ref):
  dinv = dinv_ref[...]
  out1 = jnp.maximum((p0_ref[...] + p1_ref[...]) * dinv + b1_ref[...], 0.0)
  h2_ref[...] = jnp.dot(out1, w2_ref[...],
                        preferred_element_type=jnp.float32) * dinv


def _tc3_body(q0_ref, q1_ref, dinv_ref, b2_ref, o_ref):
  s = (q0_ref[...] + q1_ref[...]) * dinv_ref[...] + b2_ref[...]
  lane = lax.broadcasted_iota(jnp.int32, s.shape, 1)
  mask = lane < 40
  s = jnp.where(mask, s, -jnp.inf)
  m = jnp.max(s, axis=1, keepdims=True)
  e = jnp.where(mask, jnp.exp(s - m), 0.0)
  lse = jnp.log(jnp.sum(e, axis=1, keepdims=True)) + m
  o_ref[...] = s - lse


def kernel(x, edge_index, edge_weight, W1, b1, W2, b2):
  N, F = x.shape
  H = W1.shape[1]
  C = W2.shape[1]
  DP = 48
  E = edge_index.shape[1]
  E2 = E + N

  Np = ((N + BLK - 1) // BLK) * BLK
  NGT = max(16, 16 * ((E2 + 16 * NS * G - 1) // (16 * NS * G)))
  E2p = NS * NGT * G

  loop = jnp.arange(N, dtype=jnp.int32)
  row = jnp.concatenate([edge_index[0].astype(jnp.int32), loop])
  col = jnp.concatenate([edge_index[1].astype(jnp.int32), loop])
  w = jnp.concatenate([edge_weight.astype(jnp.float32),
                       jnp.ones((N,), jnp.float32)])
  pad = E2p - E2
  row = jnp.pad(row, (0, pad)).reshape(NS, NGT, G)
  col = jnp.pad(col, (0, pad)).reshape(NS, NGT, G)
  w = jnp.pad(w, (0, pad)).reshape(NS, NGT, G)
  idx = jnp.stack([row, col], axis=2)

  xp = jnp.pad(x.astype(jnp.float32), ((0, Np - N), (0, 0)))
  W2p = jnp.pad(W2.astype(jnp.float32), ((0, 0), (0, DP - C)))
  b1r = b1.astype(jnp.float32).reshape(1, H)
  b2r = jnp.pad(b2.astype(jnp.float32), (0, DP - C)).reshape(1, DP)

  deg_parts = _make_deg_kernel(Np, NGT)(col, w)
  degt = deg_parts.T

  grid = (Np // BLK,)
  h1p, dinv = pl.pallas_call(
      _tc1_body,
      grid=grid,
      in_specs=[
          pl.BlockSpec((BLK, F), lambda i: (i, 0)),
          pl.BlockSpec((F, H), lambda i: (0, 0)),
          pl.BlockSpec((BLK, NC), lambda i: (i, 0)),
      ],
      out_specs=[
          pl.BlockSpec((BLK, H), lambda i: (i, 0)),
          pl.BlockSpec((BLK, 1), lambda i: (i, 0)),
      ],
      out_shape=[
          jax.ShapeDtypeStruct((Np, H), jnp.float32),
          jax.ShapeDtypeStruct((Np, 1), jnp.float32),
      ],
  )(xp, W1.astype(jnp.float32), degt)

  p = _make_agg_kernel(Np, NGT, H, False, cache_src=False)(h1p, idx, w)

  h2p = pl.pallas_call(
      _tc2_body,
      grid=grid,
      in_specs=[
          pl.BlockSpec((BLK, H), lambda i: (i, 0)),
          pl.BlockSpec((BLK, H), lambda i: (i, 0)),
          pl.BlockSpec((BLK, 1), lambda i: (i, 0)),
          pl.BlockSpec((1, H), lambda i: (0, 0)),
          pl.BlockSpec((H, DP), lambda i: (0, 0)),
      ],
      out_specs=pl.BlockSpec((BLK, DP), lambda i: (i, 0)),
      out_shape=jax.ShapeDtypeStruct((Np, DP), jnp.float32),
  )(p[0], p[1], dinv, b1r, W2p)

  q = _make_agg_kernel(Np, NGT, DP, False)(h2p, idx, w)

  out = pl.pallas_call(
      _tc3_body,
      grid=grid,
      in_specs=[
          pl.BlockSpec((BLK, DP), lambda i: (i, 0)),
          pl.BlockSpec((BLK, DP), lambda i: (i, 0)),
          pl.BlockSpec((BLK, 1), lambda i: (i, 0)),
          pl.BlockSpec((1, DP), lambda i: (0, 0)),
      ],
      out_specs=pl.BlockSpec((BLK, DP), lambda i: (i, 0)),
      out_shape=jax.ShapeDtypeStruct((Np, DP), jnp.float32),
  )(q[0], q[1], dinv, b2r)

  return out[:N, :C]


# speedup vs baseline: 2.2919x; 2.2919x over previous
import functools
import jax
import jax.numpy as jnp
from jax import lax
from jax.experimental import pallas as pl
from jax.experimental.pallas import tpu as pltpu
from jax.experimental.pallas import tpu_sc as plsc

NC = 2
NS = 16
NW = NC * NS
L = 16
G = 128
BLK = 1024


def _lane_bcast(v, jj):
  dnums = lax.GatherDimensionNumbers(
      offset_dims=(), collapsed_slice_dims=(0,), start_index_map=(0,))
  idx = jnp.full((L, 1), jj, jnp.int32)
  return lax.gather(v, idx, dnums, slice_sizes=(1,),
                    mode=lax.GatherScatterMode.PROMISE_IN_BOUNDS)


def _sc_mesh():
  return plsc.VectorSubcoreMesh(
      core_axis_name="c", subcore_axis_name="s", num_cores=NC,
      num_subcores=NS)


def _make_deg_kernel(Np, NGrp):
  @functools.partial(
      pl.kernel,
      out_type=jax.ShapeDtypeStruct((NC, Np), jnp.float32),
      mesh=_sc_mesh(),
      scratch_types=[
          pltpu.VMEM((NGrp, G), jnp.int32),
          pltpu.VMEM((NGrp, G), jnp.float32),
          pltpu.VMEM((Np // NS,), jnp.float32),
          pltpu.VMEM_SHARED((Np,), jnp.float32),
      ],
  )
  def deg_kernel(col_hbm, w_hbm, out_hbm, colv, wv, tmpv, deg_sh):
    cid = lax.axis_index("c")
    sid = lax.axis_index("s")
    wid = sid * NC + cid
    rows = Np // NS
    base = sid * rows

    @pl.loop(0, rows // L)
    def _zero(i):
      tmpv[pl.ds(i * L, L)] = jnp.zeros((L,), jnp.float32)

    pltpu.sync_copy(tmpv, deg_sh.at[pl.ds(base, rows)])
    plsc.subcore_barrier()

    pltpu.sync_copy(col_hbm.at[wid], colv)
    pltpu.sync_copy(w_hbm.at[wid], wv)

    @pl.loop(0, NGrp)
    def _acc(g):
      pltpu.sync_copy(wv.at[g], deg_sh.at[colv.at[g]], add=True)

    plsc.subcore_barrier()
    pltpu.sync_copy(deg_sh.at[pl.ds(base, rows)], tmpv)
    pltpu.sync_copy(tmpv, out_hbm.at[cid, pl.ds(base, rows)])

  return deg_kernel


def _make_agg_kernel(Np, NGrp, D, cache_src=False):
  ZR = 32
  assert NGrp % 2 == 0 and NGrp >= 4

  scratch = [
      pltpu.VMEM((2, 2, G), jnp.int32),
      pltpu.VMEM((2, G), jnp.float32),
      pltpu.VMEM((2, G, D), jnp.float32),
      pltpu.VMEM((ZR, D), jnp.float32),
      pltpu.VMEM_SHARED((Np, D), jnp.float32),
  ]
  if cache_src:
    scratch.append(pltpu.VMEM_SHARED((Np, D), jnp.float32))
  scratch += [pltpu.SemaphoreType.DMA] * 4

  @functools.partial(
      pl.kernel,
      out_type=jax.ShapeDtypeStruct((NC, Np, D), jnp.float32),
      mesh=_sc_mesh(),
      scratch_types=scratch,
      compiler_params=pltpu.CompilerParams(use_tc_tiling_on_sc=False),
  )
  def agg_kernel(h_hbm, idx_hbm, w_hbm, out_hbm,
                 cbuf, wbuf, rbuf, zv, acc_sh, *rest):
    if cache_src:
      src_ref = rest[0]
      semi0, semi1, semg0, semg1 = rest[1:]
    else:
      src_ref = h_hbm
      semi0, semi1, semg0, semg1 = rest
    cid = lax.axis_index("c")
    sid = lax.axis_index("s")
    wid = sid * NC + cid
    rows = Np // NS
    base = sid * rows
    semi = (semi0, semi1)
    semg = (semg0, semg1)

    if cache_src:
      for j in range(rows // ZR):
        sl = pl.ds(base + j * ZR, ZR)
        pltpu.sync_copy(h_hbm.at[sl], zv)
        pltpu.sync_copy(zv, src_ref.at[sl])

    @pl.loop(0, ZR)
    def _zrow(r):
      for c in range(D // L):
        zv[r, pl.ds(c * L, L)] = jnp.zeros((L,), jnp.float32)

    for j in range(rows // ZR):
      pltpu.sync_copy(zv, acc_sh.at[pl.ds(base + j * ZR, ZR)])
    plsc.subcore_barrier()

    def idx_start(g, b):
      pltpu.async_copy(idx_hbm.at[wid, g], cbuf.at[b], semi[b])
      pltpu.async_copy(w_hbm.at[wid, g], wbuf.at[b], semi[b])

    def idx_wait(g, b):
      pltpu.make_async_copy(idx_hbm.at[wid, g], cbuf.at[b], semi[b]).wait()
      pltpu.make_async_copy(w_hbm.at[wid, g], wbuf.at[b], semi[b]).wait()

    def gather_start(b):
      pltpu.async_copy(src_ref.at[cbuf.at[b, 0]], rbuf.at[b], semg[b])

    def gather_wait(b):
      pltpu.make_async_copy(src_ref.at[cbuf.at[b, 0]], rbuf.at[b],
                            semg[b]).wait()

    def process(b):
      gather_wait(b)

      @pl.loop(0, G // L)
      def _e16(t):
        vw = wbuf[b, pl.ds(t * L, L)]
        for jj in range(L):
          sv = _lane_bcast(vw, jj)
          r = t * L + jj
          for c in range(D // L):
            rbuf[b, r, pl.ds(c * L, L)] = rbuf[b, r, pl.ds(c * L, L)] * sv

      pltpu.sync_copy(rbuf.at[b], acc_sh.at[cbuf.at[b, 1]], add=True)

    idx_start(0, 0)
    idx_wait(0, 0)
    gather_start(0)
    idx_start(1, 1)
    idx_wait(1, 1)

    @pl.loop(0, NGrp - 2, step=2)
    def _pipe(g):
      gather_start(1)
      process(0)
      idx_start(g + 2, 0)
      idx_wait(g + 2, 0)
      gather_start(0)
      process(1)
      idx_start(g + 3, 1)
      idx_wait(g + 3, 1)

    gather_start(1)
    process(0)
    process(1)

    plsc.subcore_barrier()
    for j in range(rows // ZR):
      sl = pl.ds(base + j * ZR, ZR)
      pltpu.sync_copy(acc_sh.at[sl], zv)
      pltpu.sync_copy(zv, out_hbm.at[cid, sl])

  return agg_kernel


def _tc1_body(x_ref, w1_ref, degt_ref, h_ref, dinv_ref):
  deg = jnp.sum(degt_ref[...], axis=1, keepdims=True)
  dinv = jnp.where(deg > 0, lax.rsqrt(deg), 0.0)
  h = jnp.dot(x_ref[...], w1_ref[...], preferred_element_type=jnp.float32)
  h_ref[...] = h * dinv
  dinv_ref[...] = dinv


def _tc2_body(p0_ref, p1_ref, dinv_ref, b1_ref, w2_ref, h2_ref):
  dinv = dinv_ref[...]
  out1 = jnp.maximum((p0_ref[...] + p1_ref[...]) * dinv + b1_ref[...], 0.0)
  h2_ref[...] = jnp.dot(out1, w2_ref[...],
                        preferred_element_type=jnp.float32) * dinv


def _tc3_body(q0_ref, q1_ref, dinv_ref, b2_ref, o_ref):
  s = (q0_ref[...] + q1_ref[...]) * dinv_ref[...] + b2_ref[...]
  lane = lax.broadcasted_iota(jnp.int32, s.shape, 1)
  mask = lane < 40
  s = jnp.where(mask, s, -jnp.inf)
  m = jnp.max(s, axis=1, keepdims=True)
  e = jnp.where(mask, jnp.exp(s - m), 0.0)
  lse = jnp.log(jnp.sum(e, axis=1, keepdims=True)) + m
  o_ref[...] = s - lse


def kernel(x, edge_index, edge_weight, W1, b1, W2, b2):
  N, F = x.shape
  H = W1.shape[1]
  C = W2.shape[1]
  DP = 48
  E = edge_index.shape[1]
  E2 = E + N

  Np = ((N + BLK - 1) // BLK) * BLK
  NGrp = max(4, 2 * ((E2 + 2 * NW * G - 1) // (2 * NW * G)))
  E2p = NW * NGrp * G

  loop = jnp.arange(N, dtype=jnp.int32)
  row = jnp.concatenate([edge_index[0].astype(jnp.int32), loop])
  col = jnp.concatenate([edge_index[1].astype(jnp.int32), loop])
  w = jnp.concatenate([edge_weight.astype(jnp.float32),
                       jnp.ones((N,), jnp.float32)])
  pad = E2p - E2
  row = jnp.pad(row, (0, pad)).reshape(NW, NGrp, G)
  col = jnp.pad(col, (0, pad)).reshape(NW, NGrp, G)
  w = jnp.pad(w, (0, pad)).reshape(NW, NGrp, G)
  idx = jnp.stack([row, col], axis=2)

  xp = jnp.pad(x.astype(jnp.float32), ((0, Np - N), (0, 0)))
  W2p = jnp.pad(W2.astype(jnp.float32), ((0, 0), (0, DP - C)))
  b1r = b1.astype(jnp.float32).reshape(1, H)
  b2r = jnp.pad(b2.astype(jnp.float32), (0, DP - C)).reshape(1, DP)

  deg_parts = _make_deg_kernel(Np, NGrp)(col, w)
  degt = deg_parts.T

  grid = (Np // BLK,)
  h1p, dinv = pl.pallas_call(
      _tc1_body,
      grid=grid,
      in_specs=[
          pl.BlockSpec((BLK, F), lambda i: (i, 0)),
          pl.BlockSpec((F, H), lambda i: (0, 0)),
          pl.BlockSpec((BLK, NC), lambda i: (i, 0)),
      ],
      out_specs=[
          pl.BlockSpec((BLK, H), lambda i: (i, 0)),
          pl.BlockSpec((BLK, 1), lambda i: (i, 0)),
      ],
      out_shape=[
          jax.ShapeDtypeStruct((Np, H), jnp.float32),
          jax.ShapeDtypeStruct((Np, 1), jnp.float32),
      ],
  )(xp, W1.astype(jnp.float32), degt)

  p = _make_agg_kernel(Np, NGrp, H)(h1p, idx, w)

  h2p = pl.pallas_call(
      _tc2_body,
      grid=grid,
      in_specs=[
          pl.BlockSpec((BLK, H), lambda i: (i, 0)),
          pl.BlockSpec((BLK, H), lambda i: (i, 0)),
          pl.BlockSpec((BLK, 1), lambda i: (i, 0)),
          pl.BlockSpec((1, H), lambda i: (0, 0)),
          pl.BlockSpec((H, DP), lambda i: (0, 0)),
      ],
      out_specs=pl.BlockSpec((BLK, DP), lambda i: (i, 0)),
      out_shape=jax.ShapeDtypeStruct((Np, DP), jnp.float32),
  )(p[0], p[1], dinv, b1r, W2p)

  q = _make_agg_kernel(Np, NGrp, DP, cache_src=True)(h2p, idx, w)

  out = pl.pallas_call(
      _tc3_body,
      grid=grid,
      in_specs=[
          pl.BlockSpec((BLK, DP), lambda i: (i, 0)),
          pl.BlockSpec((BLK, DP), lambda i: (i, 0)),
          pl.BlockSpec((BLK, 1), lambda i: (i, 0)),
          pl.BlockSpec((1, DP), lambda i: (0, 0)),
      ],
      out_specs=pl.BlockSpec((BLK, DP), lambda i: (i, 0)),
      out_shape=jax.ShapeDtypeStruct((Np, DP), jnp.float32),
  )(q[0], q[1], dinv, b2r)

  return out[:N, :C]


# speedup vs baseline: 2.3337x; 1.0182x over previous
import functools
import jax
import jax.numpy as jnp
from jax import lax
from jax.experimental import pallas as pl
from jax.experimental.pallas import tpu as pltpu
from jax.experimental.pallas import tpu_sc as plsc

NC = 2
NS = 16
NW = NC * NS
L = 16
G = 128
BLK = 1024


def _lane_bcast(v, jj):
  dnums = lax.GatherDimensionNumbers(
      offset_dims=(), collapsed_slice_dims=(0,), start_index_map=(0,))
  idx = jnp.full((L, 1), jj, jnp.int32)
  return lax.gather(v, idx, dnums, slice_sizes=(1,),
                    mode=lax.GatherScatterMode.PROMISE_IN_BOUNDS)


def _sc_mesh():
  return plsc.VectorSubcoreMesh(
      core_axis_name="c", subcore_axis_name="s", num_cores=NC,
      num_subcores=NS)


def _make_deg_kernel(Np, NGrp):
  @functools.partial(
      pl.kernel,
      out_type=jax.ShapeDtypeStruct((NC, Np), jnp.float32),
      mesh=_sc_mesh(),
      scratch_types=[
          pltpu.VMEM((NGrp, G), jnp.int32),
          pltpu.VMEM((NGrp, G), jnp.float32),
          pltpu.VMEM((Np // NS,), jnp.float32),
          pltpu.VMEM_SHARED((Np,), jnp.float32),
      ],
  )
  def deg_kernel(col_hbm, w_hbm, out_hbm, colv, wv, tmpv, deg_sh):
    cid = lax.axis_index("c")
    sid = lax.axis_index("s")
    wid = sid * NC + cid
    rows = Np // NS
    base = sid * rows

    @pl.loop(0, rows // L)
    def _zero(i):
      tmpv[pl.ds(i * L, L)] = jnp.zeros((L,), jnp.float32)

    pltpu.sync_copy(tmpv, deg_sh.at[pl.ds(base, rows)])
    plsc.subcore_barrier()

    pltpu.sync_copy(col_hbm.at[wid], colv)
    pltpu.sync_copy(w_hbm.at[wid], wv)

    @pl.loop(0, NGrp)
    def _acc(g):
      pltpu.sync_copy(wv.at[g], deg_sh.at[colv.at[g]], add=True)

    plsc.subcore_barrier()
    pltpu.sync_copy(deg_sh.at[pl.ds(base, rows)], tmpv)
    pltpu.sync_copy(tmpv, out_hbm.at[cid, pl.ds(base, rows)])

  return deg_kernel


def _make_agg_kernel(Np, NGrp, D, cache_src=False):
  ZR = 32
  assert NGrp % 2 == 0 and NGrp >= 4

  scratch = [
      pltpu.VMEM((2, 2, G), jnp.int32),
      pltpu.VMEM((2, G), jnp.float32),
      pltpu.VMEM((2, G, D), jnp.float32),
      pltpu.VMEM((ZR, D), jnp.float32),
      pltpu.VMEM_SHARED((Np, D), jnp.float32),
  ]
  if cache_src:
    scratch.append(pltpu.VMEM_SHARED((Np, D), jnp.float32))
  scratch += [pltpu.SemaphoreType.DMA] * 4

  @functools.partial(
      pl.kernel,
      out_type=jax.ShapeDtypeStruct((NC, Np, D), jnp.float32),
      mesh=_sc_mesh(),
      scratch_types=scratch,
      compiler_params=pltpu.CompilerParams(use_tc_tiling_on_sc=False),
  )
  def agg_kernel(h_hbm, idx_hbm, w_hbm, out_hbm,
                 cbuf, wbuf, rbuf, zv, acc_sh, *rest):
    if cache_src:
      src_ref = rest[0]
      semi0, semi1, semg0, semg1 = rest[1:]
    else:
      src_ref = h_hbm
      semi0, semi1, semg0, semg1 = rest
    cid = lax.axis_index("c")
    sid = lax.axis_index("s")
    wid = sid * NC + cid
    rows = Np // NS
    base = sid * rows
    semi = (semi0, semi1)
    semg = (semg0, semg1)

    if cache_src:
      sl = pl.ds(base, rows)
      pltpu.sync_copy(h_hbm.at[sl], src_ref.at[sl])

    @pl.loop(0, ZR)
    def _zrow(r):
      for c in range(D // L):
        zv[r, pl.ds(c * L, L)] = jnp.zeros((L,), jnp.float32)

    for j in range(rows // ZR):
      pltpu.sync_copy(zv, acc_sh.at[pl.ds(base + j * ZR, ZR)])
    plsc.subcore_barrier()

    def idx_start(g, b):
      pltpu.async_copy(idx_hbm.at[wid, g], cbuf.at[b], semi[b])
      pltpu.async_copy(w_hbm.at[wid, g], wbuf.at[b], semi[b])

    def idx_wait(g, b):
      pltpu.make_async_copy(idx_hbm.at[wid, g], cbuf.at[b], semi[b]).wait()
      pltpu.make_async_copy(w_hbm.at[wid, g], wbuf.at[b], semi[b]).wait()

    def gather_start(b):
      pltpu.async_copy(src_ref.at[cbuf.at[b, 0]], rbuf.at[b], semg[b])

    def gather_wait(b):
      pltpu.make_async_copy(src_ref.at[cbuf.at[b, 0]], rbuf.at[b],
                            semg[b]).wait()

    def process(b):
      gather_wait(b)

      @pl.loop(0, G // L)
      def _e16(t):
        vw = wbuf[b, pl.ds(t * L, L)]
        for jj in range(L):
          sv = _lane_bcast(vw, jj)
          r = t * L + jj
          for c in range(D // L):
            rbuf[b, r, pl.ds(c * L, L)] = rbuf[b, r, pl.ds(c * L, L)] * sv

      pltpu.sync_copy(rbuf.at[b], acc_sh.at[cbuf.at[b, 1]], add=True)

    idx_start(0, 0)
    idx_wait(0, 0)
    gather_start(0)
    idx_start(1, 1)
    idx_wait(1, 1)

    @pl.loop(0, NGrp - 2, step=2)
    def _pipe(g):
      gather_start(1)
      process(0)
      idx_start(g + 2, 0)
      idx_wait(g + 2, 0)
      gather_start(0)
      process(1)
      idx_start(g + 3, 1)
      idx_wait(g + 3, 1)

    gather_start(1)
    process(0)
    process(1)

    plsc.subcore_barrier()
    for j in range(rows // ZR):
      sl = pl.ds(base + j * ZR, ZR)
      pltpu.sync_copy(acc_sh.at[sl], zv)
      pltpu.sync_copy(zv, out_hbm.at[cid, sl])

  return agg_kernel


def _tc1_body(x_ref, w1_ref, degt_ref, h_ref, dinv_ref):
  deg = jnp.sum(degt_ref[...], axis=1, keepdims=True)
  dinv = jnp.where(deg > 0, lax.rsqrt(deg), 0.0)
  h = jnp.dot(x_ref[...], w1_ref[...], preferred_element_type=jnp.float32)
  h_ref[...] = h * dinv
  dinv_ref[...] = dinv


def _tc2_body(p0_ref, p1_ref, dinv_ref, b1_ref, w2_ref, h2_ref):
  dinv = dinv_ref[...]
  out1 = jnp.maximum((p0_ref[...] + p1_ref[...]) * dinv + b1_ref[...], 0.0)
  h2_ref[...] = jnp.dot(out1, w2_ref[...],
                        preferred_element_type=jnp.float32) * dinv


def _tc3_body(q0_ref, q1_ref, dinv_ref, b2_ref, o_ref):
  s = (q0_ref[...] + q1_ref[...]) * dinv_ref[...] + b2_ref[...]
  lane = lax.broadcasted_iota(jnp.int32, s.shape, 1)
  mask = lane < 40
  s = jnp.where(mask, s, -jnp.inf)
  m = jnp.max(s, axis=1, keepdims=True)
  e = jnp.where(mask, jnp.exp(s - m), 0.0)
  lse = jnp.log(jnp.sum(e, axis=1, keepdims=True)) + m
  o_ref[...] = s - lse


def kernel(x, edge_index, edge_weight, W1, b1, W2, b2):
  N, F = x.shape
  H = W1.shape[1]
  C = W2.shape[1]
  DP = 48
  E = edge_index.shape[1]
  E2 = E + N

  Np = ((N + BLK - 1) // BLK) * BLK
  NGrp = max(4, 2 * ((E2 + 2 * NW * G - 1) // (2 * NW * G)))
  E2p = NW * NGrp * G

  loop = jnp.arange(N, dtype=jnp.int32)
  row = jnp.concatenate([edge_index[0].astype(jnp.int32), loop])
  col = jnp.concatenate([edge_index[1].astype(jnp.int32), loop])
  w = jnp.concatenate([edge_weight.astype(jnp.float32),
                       jnp.ones((N,), jnp.float32)])
  pad = E2p - E2
  row = jnp.pad(row, (0, pad)).reshape(NW, NGrp, G)
  col = jnp.pad(col, (0, pad)).reshape(NW, NGrp, G)
  w = jnp.pad(w, (0, pad)).reshape(NW, NGrp, G)
  idx = jnp.stack([row, col], axis=2)

  xp = jnp.pad(x.astype(jnp.float32), ((0, Np - N), (0, 0)))
  W2p = jnp.pad(W2.astype(jnp.float32), ((0, 0), (0, DP - C)))
  b1r = b1.astype(jnp.float32).reshape(1, H)
  b2r = jnp.pad(b2.astype(jnp.float32), (0, DP - C)).reshape(1, DP)

  deg_parts = _make_deg_kernel(Np, NGrp)(col, w)
  degt = deg_parts.T

  grid = (Np // BLK,)
  h1p, dinv = pl.pallas_call(
      _tc1_body,
      grid=grid,
      in_specs=[
          pl.BlockSpec((BLK, F), lambda i: (i, 0)),
          pl.BlockSpec((F, H), lambda i: (0, 0)),
          pl.BlockSpec((BLK, NC), lambda i: (i, 0)),
      ],
      out_specs=[
          pl.BlockSpec((BLK, H), lambda i: (i, 0)),
          pl.BlockSpec((BLK, 1), lambda i: (i, 0)),
      ],
      out_shape=[
          jax.ShapeDtypeStruct((Np, H), jnp.float32),
          jax.ShapeDtypeStruct((Np, 1), jnp.float32),
      ],
  )(xp, W1.astype(jnp.float32), degt)

  p = _make_agg_kernel(Np, NGrp, H)(h1p, idx, w)

  h2p = pl.pallas_call(
      _tc2_body,
      grid=grid,
      in_specs=[
          pl.BlockSpec((BLK, H), lambda i: (i, 0)),
          pl.BlockSpec((BLK, H), lambda i: (i, 0)),
          pl.BlockSpec((BLK, 1), lambda i: (i, 0)),
          pl.BlockSpec((1, H), lambda i: (0, 0)),
          pl.BlockSpec((H, DP), lambda i: (0, 0)),
      ],
      out_specs=pl.BlockSpec((BLK, DP), lambda i: (i, 0)),
      out_shape=jax.ShapeDtypeStruct((Np, DP), jnp.float32),
  )(p[0], p[1], dinv, b1r, W2p)

  q = _make_agg_kernel(Np, NGrp, DP, cache_src=True)(h2p, idx, w)

  out = pl.pallas_call(
      _tc3_body,
      grid=grid,
      in_specs=[
          pl.BlockSpec((BLK, DP), lambda i: (i, 0)),
          pl.BlockSpec((BLK, DP), lambda i: (i, 0)),
          pl.BlockSpec((BLK, 1), lambda i: (i, 0)),
          pl.BlockSpec((1, DP), lambda i: (0, 0)),
      ],
      out_specs=pl.BlockSpec((BLK, DP), lambda i: (i, 0)),
      out_shape=jax.ShapeDtypeStruct((Np, DP), jnp.float32),
  )(q[0], q[1], dinv, b2r)

  return out[:N, :C]
